# baseline (speedup 1.0000x reference)
import functools

import jax
import jax.numpy as jnp
from jax import lax
from jax.experimental import pallas as pl
from jax.experimental.pallas import tpu as pltpu
from jax.experimental.pallas import tpu_sc as plsc

N_NODES = 10000
IN_CH = 128
OUT_CH = 128

NC = 2
NS = 16
NW = NC * NS
CHUNK = 128
NPAD = 10240
ROWS_PER_TILE = NPAD // NS

N_EDGES = 320000
N_CHUNKS_W = 80
E_PER_W = N_CHUNKS_W * CHUNK
EPAD = E_PER_W * NW
E_PER_C = EPAD // NC

N_CHUNKS_PAIR = 160
NCH0 = 42
NCH1 = N_CHUNKS_PAIR - NCH0

BLK = 1024
GRID = NPAD // BLK

_mesh = lambda: plsc.VectorSubcoreMesh(
    core_axis_name="c", subcore_axis_name="s", num_cores=NC, num_subcores=NS)


@functools.partial(
    pl.kernel,
    out_type=jax.ShapeDtypeStruct((NC, NPAD), jnp.float32),
    mesh=_mesh(),
    scratch_types=[
        pltpu.VMEM((CHUNK,), jnp.int32),
        pltpu.VMEM((CHUNK,), jnp.float32),
        pltpu.VMEM_SHARED((NPAD,), jnp.float32),
    ],
)
def _deg_kernel(row_hbm, hist_hbm, idxv, onesv, acc):
    c = lax.axis_index("c")
    s = lax.axis_index("s")

    for k in range(CHUNK // 16):
        onesv[pl.ds(k * 16, 16)] = jnp.zeros((16,), jnp.float32)
    base_r = s * ROWS_PER_TILE
    @pl.loop(0, ROWS_PER_TILE // CHUNK)
    def _zero(i):
        pltpu.sync_copy(onesv, acc.at[pl.ds(base_r + i * CHUNK, CHUNK)])
    for k in range(CHUNK // 16):
        onesv[pl.ds(k * 16, 16)] = jnp.ones((16,), jnp.float32)
    plsc.subcore_barrier()

    base_e = c * E_PER_C + s * E_PER_W
    @pl.loop(0, N_CHUNKS_W)
    def _hist(j):
        pltpu.sync_copy(row_hbm.at[pl.ds(base_e + j * CHUNK, CHUNK)], idxv)
        pltpu.sync_copy(onesv, acc.at[idxv], add=True)
    plsc.subcore_barrier()

    @pl.loop(0, ROWS_PER_TILE // CHUNK)
    def _wb(i):
        off = base_r + i * CHUNK
        pltpu.sync_copy(acc.at[pl.ds(off, CHUNK)], onesv)
        pltpu.sync_copy(onesv, hist_hbm.at[c, pl.ds(off, CHUNK)])


@functools.partial(
    pl.kernel,
    out_type=jax.ShapeDtypeStruct((NC, NPAD, OUT_CH), jnp.float32),
    mesh=_mesh(),
    scratch_types=[
        pltpu.VMEM((CHUNK,), jnp.int32),
        pltpu.VMEM((CHUNK,), jnp.int32),
        pltpu.VMEM((2, CHUNK), jnp.int32),
        pltpu.VMEM((CHUNK, OUT_CH), jnp.float32),
        pltpu.VMEM((CHUNK, OUT_CH), jnp.float32),
        pltpu.VMEM((8, OUT_CH), jnp.float32),
        pltpu.VMEM_SHARED((NPAD, OUT_CH), jnp.float32),
        pltpu.SemaphoreType.DMA,
        pltpu.SemaphoreType.DMA,
    ],
)
def _edge_kernel(sup_hbm, col_hbm, row_hbm, out_hbm,
                 colv0, colv1, rowv, buf0, buf1, ztile, acc, sem0, sem1):
    c = lax.axis_index("c")
    s = lax.axis_index("s")

    for r in range(8):
        for k in range(OUT_CH // 16):
            ztile[r, pl.ds(k * 16, 16)] = jnp.zeros((16,), jnp.float32)
    base_r = s * ROWS_PER_TILE
    @pl.loop(0, ROWS_PER_TILE // 8)
    def _zero(i):
        pltpu.sync_copy(ztile, acc.at[pl.ds(base_r + i * 8, 8)])
    plsc.subcore_barrier()

    n_my = jnp.where(c == 0, NCH0, NCH1)
    base_e = jnp.where(c == 0, s * NCH0, NS * NCH0 + s * NCH1) * CHUNK

    colvs = (colv0, colv1)
    bufs = (buf0, buf1)
    sems = (sem0, sem1)
    pltpu.sync_copy(col_hbm.at[pl.ds(base_e, CHUNK)], colv0)
    pltpu.async_copy(sup_hbm.at[colv0], buf0, sem0)
    @pl.loop(0, n_my // 2)
    def _pair(i):
        j0 = 2 * i
        for p in range(2):
            j = j0 + p
            jn = lax.rem(j + 1, n_my)
            pltpu.sync_copy(col_hbm.at[pl.ds(base_e + jn * CHUNK, CHUNK)],
                            colvs[1 - p])
            pltpu.async_copy(sup_hbm.at[colvs[1 - p]], bufs[1 - p], sems[1 - p])
            pltpu.sync_copy(row_hbm.at[pl.ds(base_e + j * CHUNK, CHUNK)],
                            rowv.at[p])
            pltpu.make_async_copy(sup_hbm.at[colvs[p]], bufs[p], sems[p]).wait()
            pltpu.sync_copy(bufs[p], acc.at[rowv.at[p]], add=True)
    pltpu.make_async_copy(sup_hbm.at[colv0], buf0, sem0).wait()
    plsc.subcore_barrier()

    @pl.loop(0, ROWS_PER_TILE // CHUNK)
    def _wb(i):
        off = base_r + i * CHUNK
        pltpu.sync_copy(acc.at[pl.ds(off, CHUNK)], buf0)
        pltpu.sync_copy(buf0, out_hbm.at[c, pl.ds(off, CHUNK)])


def _support_body(x_ref, wt_ref, b_ref, h0_ref, h1_ref, sup_ref, dis_ref):
    deg = 1.0 + h0_ref[...] + h1_ref[...]
    dis = lax.rsqrt(deg)
    s = jnp.dot(x_ref[...], wt_ref[...],
                preferred_element_type=jnp.float32) + b_ref[...]
    sup_ref[...] = dis * s
    dis_ref[...] = dis


def _support_call(x_pad, wt, b2, h0, h1):
    return pl.pallas_call(
        _support_body,
        grid=(GRID,),
        in_specs=[
            pl.BlockSpec((BLK, IN_CH), lambda i: (i, 0)),
            pl.BlockSpec((IN_CH, OUT_CH), lambda i: (0, 0)),
            pl.BlockSpec((1, OUT_CH), lambda i: (0, 0)),
            pl.BlockSpec((BLK, 1), lambda i: (i, 0)),
            pl.BlockSpec((BLK, 1), lambda i: (i, 0)),
        ],
        out_specs=[
            pl.BlockSpec((BLK, OUT_CH), lambda i: (i, 0)),
            pl.BlockSpec((BLK, 1), lambda i: (i, 0)),
        ],
        out_shape=[
            jax.ShapeDtypeStruct((NPAD, OUT_CH), jnp.float32),
            jax.ShapeDtypeStruct((NPAD, 1), jnp.float32),
        ],
    )(x_pad, wt, b2, h0, h1)


def _combine_body(p0_ref, p1_ref, sup_ref, dis_ref, out_ref):
    out_ref[...] = dis_ref[...] * (p0_ref[...] + p1_ref[...] + sup_ref[...])


def _combine_call(p0, p1, sup, dis):
    return pl.pallas_call(
        _combine_body,
        grid=(GRID,),
        in_specs=[
            pl.BlockSpec((BLK, OUT_CH), lambda i: (i, 0)),
            pl.BlockSpec((BLK, OUT_CH), lambda i: (i, 0)),
            pl.BlockSpec((BLK, OUT_CH), lambda i: (i, 0)),
            pl.BlockSpec((BLK, 1), lambda i: (i, 0)),
        ],
        out_specs=pl.BlockSpec((BLK, OUT_CH), lambda i: (i, 0)),
        out_shape=jax.ShapeDtypeStruct((NPAD, OUT_CH), jnp.float32),
    )(p0, p1, sup, dis)


def kernel(x, edge_index, W, b):
    ei = edge_index.astype(jnp.int32)
    row = jnp.pad(ei[0], (0, EPAD - N_EDGES), constant_values=N_NODES)
    col = jnp.pad(ei[1], (0, EPAD - N_EDGES), constant_values=0)

    hist = _deg_kernel(row)
    h0 = hist[0].reshape(NPAD, 1)
    h1 = hist[1].reshape(NPAD, 1)

    x_pad = jnp.pad(x, ((0, NPAD - N_NODES), (0, 0)))
    wt = W.T
    b2 = b.reshape(1, OUT_CH)
    sup, dis = _support_call(x_pad, wt, b2, h0, h1)

    partials = _edge_kernel(sup, col, row)
    out = _combine_call(partials[0], partials[1], sup, dis)
    return out[:N_NODES]


# speedup vs baseline: 1.0902x; 1.0902x over previous
import functools

import jax
import jax.numpy as jnp
from jax import lax
from jax.experimental import pallas as pl
from jax.experimental.pallas import tpu as pltpu
from jax.experimental.pallas import tpu_sc as plsc

N_NODES = 10000
IN_CH = 128
OUT_CH = 128

NC = 2
NS = 16
NW = NC * NS
CHUNK = 128
NPAD = 10240
ROWS_PER_TILE = NPAD // NS

N_EDGES = 320000
N_CHUNKS_W = 80
E_PER_W = N_CHUNKS_W * CHUNK
EPAD = E_PER_W * NW
E_PER_C = EPAD // NC

N_CHUNKS_PAIR = 160
NCH0 = 118
NCH1 = N_CHUNKS_PAIR - NCH0

BLK = 1024
GRID = NPAD // BLK

_mesh = lambda: plsc.VectorSubcoreMesh(
    core_axis_name="c", subcore_axis_name="s", num_cores=NC, num_subcores=NS)


@functools.partial(
    pl.kernel,
    out_type=jax.ShapeDtypeStruct((NC, NPAD), jnp.float32),
    mesh=_mesh(),
    scratch_types=[
        pltpu.VMEM((CHUNK,), jnp.int32),
        pltpu.VMEM((CHUNK,), jnp.float32),
        pltpu.VMEM_SHARED((NPAD,), jnp.float32),
    ],
)
def _deg_kernel(row_hbm, hist_hbm, idxv, onesv, acc):
    c = lax.axis_index("c")
    s = lax.axis_index("s")

    for k in range(CHUNK // 16):
        onesv[pl.ds(k * 16, 16)] = jnp.zeros((16,), jnp.float32)
    base_r = s * ROWS_PER_TILE
    @pl.loop(0, ROWS_PER_TILE // CHUNK)
    def _zero(i):
        pltpu.sync_copy(onesv, acc.at[pl.ds(base_r + i * CHUNK, CHUNK)])
    for k in range(CHUNK // 16):
        onesv[pl.ds(k * 16, 16)] = jnp.ones((16,), jnp.float32)
    plsc.subcore_barrier()

    base_e = c * E_PER_C + s * E_PER_W
    @pl.loop(0, N_CHUNKS_W)
    def _hist(j):
        pltpu.sync_copy(row_hbm.at[pl.ds(base_e + j * CHUNK, CHUNK)], idxv)
        pltpu.sync_copy(onesv, acc.at[idxv], add=True)
    plsc.subcore_barrier()

    @pl.loop(0, ROWS_PER_TILE // CHUNK)
    def _wb(i):
        off = base_r + i * CHUNK
        pltpu.sync_copy(acc.at[pl.ds(off, CHUNK)], onesv)
        pltpu.sync_copy(onesv, hist_hbm.at[c, pl.ds(off, CHUNK)])


@functools.partial(
    pl.kernel,
    out_type=jax.ShapeDtypeStruct((NC, NPAD, OUT_CH), jnp.float32),
    mesh=_mesh(),
    scratch_types=[
        pltpu.VMEM((CHUNK,), jnp.int32),
        pltpu.VMEM((CHUNK,), jnp.int32),
        pltpu.VMEM((2, CHUNK), jnp.int32),
        pltpu.VMEM((CHUNK, OUT_CH), jnp.float32),
        pltpu.VMEM((CHUNK, OUT_CH), jnp.float32),
        pltpu.VMEM((8, OUT_CH), jnp.float32),
        pltpu.VMEM_SHARED((NPAD, OUT_CH), jnp.float32),
        pltpu.SemaphoreType.DMA,
        pltpu.SemaphoreType.DMA,
    ],
)
def _edge_kernel(sup_hbm, col_hbm, row_hbm, out_hbm,
                 colv0, colv1, rowv, buf0, buf1, ztile, acc, sem0, sem1):
    c = lax.axis_index("c")
    s = lax.axis_index("s")

    for r in range(8):
        for k in range(OUT_CH // 16):
            ztile[r, pl.ds(k * 16, 16)] = jnp.zeros((16,), jnp.float32)
    base_r = s * ROWS_PER_TILE
    @pl.loop(0, ROWS_PER_TILE // 8)
    def _zero(i):
        pltpu.sync_copy(ztile, acc.at[pl.ds(base_r + i * 8, 8)])
    plsc.subcore_barrier()

    n_my = jnp.where(c == 0, NCH0, NCH1)
    base_e = jnp.where(c == 0, s * NCH0, NS * NCH0 + s * NCH1) * CHUNK

    colvs = (colv0, colv1)
    bufs = (buf0, buf1)
    sems = (sem0, sem1)
    pltpu.sync_copy(col_hbm.at[pl.ds(base_e, CHUNK)], colv0)
    pltpu.async_copy(sup_hbm.at[colv0], buf0, sem0)
    @pl.loop(0, n_my // 2)
    def _pair(i):
        j0 = 2 * i
        for p in range(2):
            j = j0 + p
            jn = lax.rem(j + 1, n_my)
            pltpu.sync_copy(col_hbm.at[pl.ds(base_e + jn * CHUNK, CHUNK)],
                            colvs[1 - p])
            pltpu.async_copy(sup_hbm.at[colvs[1 - p]], bufs[1 - p], sems[1 - p])
            pltpu.sync_copy(row_hbm.at[pl.ds(base_e + j * CHUNK, CHUNK)],
                            rowv.at[p])
            pltpu.make_async_copy(sup_hbm.at[colvs[p]], bufs[p], sems[p]).wait()
            pltpu.sync_copy(bufs[p], acc.at[rowv.at[p]], add=True)
    pltpu.make_async_copy(sup_hbm.at[colv0], buf0, sem0).wait()
    plsc.subcore_barrier()

    @pl.loop(0, ROWS_PER_TILE // CHUNK)
    def _wb(i):
        off = base_r + i * CHUNK
        pltpu.sync_copy(acc.at[pl.ds(off, CHUNK)], buf0)
        pltpu.sync_copy(buf0, out_hbm.at[c, pl.ds(off, CHUNK)])


def _support_body(x_ref, wt_ref, b_ref, h0_ref, h1_ref, sup_ref, dis_ref):
    deg = 1.0 + h0_ref[...] + h1_ref[...]
    dis = lax.rsqrt(deg)
    s = jnp.dot(x_ref[...], wt_ref[...],
                preferred_element_type=jnp.float32) + b_ref[...]
    sup_ref[...] = dis * s
    dis_ref[...] = dis


def _support_call(x_pad, wt, b2, h0, h1):
    return pl.pallas_call(
        _support_body,
        grid=(GRID,),
        in_specs=[
            pl.BlockSpec((BLK, IN_CH), lambda i: (i, 0)),
            pl.BlockSpec((IN_CH, OUT_CH), lambda i: (0, 0)),
            pl.BlockSpec((1, OUT_CH), lambda i: (0, 0)),
            pl.BlockSpec((BLK, 1), lambda i: (i, 0)),
            pl.BlockSpec((BLK, 1), lambda i: (i, 0)),
        ],
        out_specs=[
            pl.BlockSpec((BLK, OUT_CH), lambda i: (i, 0)),
            pl.BlockSpec((BLK, 1), lambda i: (i, 0)),
        ],
        out_shape=[
            jax.ShapeDtypeStruct((NPAD, OUT_CH), jnp.float32),
            jax.ShapeDtypeStruct((NPAD, 1), jnp.float32),
        ],
    )(x_pad, wt, b2, h0, h1)


def _combine_body(p0_ref, p1_ref, sup_ref, dis_ref, out_ref):
    out_ref[...] = dis_ref[...] * (p0_ref[...] + p1_ref[...] + sup_ref[...])


def _combine_call(p0, p1, sup, dis):
    return pl.pallas_call(
        _combine_body,
        grid=(GRID,),
        in_specs=[
            pl.BlockSpec((BLK, OUT_CH), lambda i: (i, 0)),
            pl.BlockSpec((BLK, OUT_CH), lambda i: (i, 0)),
            pl.BlockSpec((BLK, OUT_CH), lambda i: (i, 0)),
            pl.BlockSpec((BLK, 1), lambda i: (i, 0)),
        ],
        out_specs=pl.BlockSpec((BLK, OUT_CH), lambda i: (i, 0)),
        out_shape=jax.ShapeDtypeStruct((NPAD, OUT_CH), jnp.float32),
    )(p0, p1, sup, dis)


def kernel(x, edge_index, W, b):
    ei = edge_index.astype(jnp.int32)
    row = jnp.pad(ei[0], (0, EPAD - N_EDGES), constant_values=N_NODES)
    col = jnp.pad(ei[1], (0, EPAD - N_EDGES), constant_values=0)

    hist = _deg_kernel(row)
    h0 = hist[0].reshape(NPAD, 1)
    h1 = hist[1].reshape(NPAD, 1)

    x_pad = jnp.pad(x, ((0, NPAD - N_NODES), (0, 0)))
    wt = W.T
    b2 = b.reshape(1, OUT_CH)
    sup, dis = _support_call(x_pad, wt, b2, h0, h1)

    partials = _edge_kernel(sup, col, row)
    out = _combine_call(partials[0], partials[1], sup, dis)
    return out[:N_NODES]


# speedup vs baseline: 1.4377x; 1.3187x over previous
import functools

import jax
import jax.numpy as jnp
from jax import lax
from jax.experimental import pallas as pl
from jax.experimental.pallas import tpu as pltpu
from jax.experimental.pallas import tpu_sc as plsc

N_NODES = 10000
IN_CH = 128
OUT_CH = 128

NC = 2
NS = 16
NW = NC * NS
CHUNK = 128
NPAD = 10240
ROWS_PER_TILE = NPAD // NS

N_EDGES = 320000
N_CHUNKS_W = 80
E_PER_W = N_CHUNKS_W * CHUNK
EPAD = E_PER_W * NW
E_PER_C = EPAD // NC

N_CHUNKS_PAIR = 160
NCH0 = 118
NCH1 = N_CHUNKS_PAIR - NCH0

BLK = 1024
GRID = NPAD // BLK

_mesh = lambda: plsc.VectorSubcoreMesh(
    core_axis_name="c", subcore_axis_name="s", num_cores=NC, num_subcores=NS)


@functools.partial(
    pl.kernel,
    out_type=jax.ShapeDtypeStruct((NC, NPAD), jnp.float32),
    mesh=_mesh(),
    scratch_types=[
        pltpu.VMEM((CHUNK,), jnp.int32),
        pltpu.VMEM((CHUNK,), jnp.float32),
        pltpu.VMEM_SHARED((NPAD,), jnp.float32),
    ],
)
def _deg_kernel(row_hbm, hist_hbm, idxv, onesv, acc):
    c = lax.axis_index("c")
    s = lax.axis_index("s")

    for k in range(CHUNK // 16):
        onesv[pl.ds(k * 16, 16)] = jnp.zeros((16,), jnp.float32)
    base_r = s * ROWS_PER_TILE
    @pl.loop(0, ROWS_PER_TILE // CHUNK)
    def _zero(i):
        pltpu.sync_copy(onesv, acc.at[pl.ds(base_r + i * CHUNK, CHUNK)])
    for k in range(CHUNK // 16):
        onesv[pl.ds(k * 16, 16)] = jnp.ones((16,), jnp.float32)
    plsc.subcore_barrier()

    base_e = c * E_PER_C + s * E_PER_W
    @pl.loop(0, N_CHUNKS_W)
    def _hist(j):
        pltpu.sync_copy(row_hbm.at[pl.ds(base_e + j * CHUNK, CHUNK)], idxv)
        pltpu.sync_copy(onesv, acc.at[idxv], add=True)
    plsc.subcore_barrier()

    @pl.loop(0, ROWS_PER_TILE // CHUNK)
    def _wb(i):
        off = base_r + i * CHUNK
        pltpu.sync_copy(acc.at[pl.ds(off, CHUNK)], onesv)
        pltpu.sync_copy(onesv, hist_hbm.at[c, pl.ds(off, CHUNK)])


@functools.partial(
    pl.kernel,
    out_type=jax.ShapeDtypeStruct((NC, NPAD, OUT_CH), jnp.float32),
    mesh=_mesh(),
    scratch_types=[
        pltpu.VMEM((CHUNK,), jnp.int32),
        pltpu.VMEM((CHUNK,), jnp.int32),
        pltpu.VMEM((2, CHUNK), jnp.int32),
        pltpu.VMEM((CHUNK, OUT_CH), jnp.float32),
        pltpu.VMEM((CHUNK, OUT_CH), jnp.float32),
        pltpu.VMEM((8, OUT_CH), jnp.float32),
        pltpu.VMEM_SHARED((NPAD, OUT_CH), jnp.float32),
        pltpu.SemaphoreType.DMA,
        pltpu.SemaphoreType.DMA,
    ],
)
def _edge_kernel(sup_hbm, col_hbm, row_hbm, out_hbm,
                 colv0, colv1, rowv, buf0, buf1, ztile, acc, sem0, sem1):
    c = lax.axis_index("c")
    s = lax.axis_index("s")

    for r in range(8):
        for k in range(OUT_CH // 16):
            ztile[r, pl.ds(k * 16, 16)] = jnp.zeros((16,), jnp.float32)
    base_r = s * ROWS_PER_TILE
    @pl.loop(0, ROWS_PER_TILE // 8)
    def _zero(i):
        pltpu.sync_copy(ztile, acc.at[pl.ds(base_r + i * 8, 8)])
    plsc.subcore_barrier()

    n_my = jnp.where(c == 0, NCH0, NCH1)
    base_e = jnp.where(c == 0, s * NCH0, NS * NCH0 + s * NCH1) * CHUNK

    colvs = (colv0, colv1)
    bufs = (buf0, buf1)
    sems = (sem0, sem1)
    pltpu.sync_copy(col_hbm.at[pl.ds(base_e, CHUNK)], colv0)
    pltpu.async_copy(sup_hbm.at[colv0], buf0, sem0)
    @pl.loop(0, n_my // 2)
    def _pair(i):
        j0 = 2 * i
        for p in range(2):
            j = j0 + p
            jn = lax.rem(j + 1, n_my)
            pltpu.sync_copy(col_hbm.at[pl.ds(base_e + jn * CHUNK, CHUNK)],
                            colvs[1 - p])
            pltpu.async_copy(sup_hbm.at[colvs[1 - p]], bufs[1 - p], sems[1 - p])
            pltpu.sync_copy(row_hbm.at[pl.ds(base_e + j * CHUNK, CHUNK)],
                            rowv.at[p])
            pltpu.make_async_copy(sup_hbm.at[colvs[p]], bufs[p], sems[p]).wait()
            pltpu.sync_copy(bufs[p], acc.at[rowv.at[p]], add=True)
    pltpu.make_async_copy(sup_hbm.at[colv0], buf0, sem0).wait()
    plsc.subcore_barrier()

    @pl.loop(0, ROWS_PER_TILE // CHUNK)
    def _wb(i):
        off = base_r + i * CHUNK
        pltpu.sync_copy(acc.at[pl.ds(off, CHUNK)], buf0)
        pltpu.sync_copy(buf0, out_hbm.at[c, pl.ds(off, CHUNK)])


HCH = OUT_CH // NC
N_CHUNKS_T = EPAD // (NS * CHUNK)


@functools.partial(
    pl.kernel,
    out_type=jax.ShapeDtypeStruct((NC, NPAD, HCH), jnp.float32),
    mesh=_mesh(),
    compiler_params=pltpu.CompilerParams(use_tc_tiling_on_sc=False),
    scratch_types=[
        pltpu.VMEM((CHUNK,), jnp.int32),
        pltpu.VMEM((CHUNK,), jnp.int32),
        pltpu.VMEM((2, CHUNK), jnp.int32),
        pltpu.VMEM((CHUNK, HCH), jnp.float32),
        pltpu.VMEM((CHUNK, HCH), jnp.float32),
        pltpu.VMEM((8, HCH), jnp.float32),
        pltpu.VMEM_SHARED((NPAD, HCH), jnp.float32),
        pltpu.VMEM_SHARED((NPAD, HCH), jnp.float32),
        pltpu.SemaphoreType.DMA,
        pltpu.SemaphoreType.DMA,
    ],
)
def _edge_kernel_cs(sup_hbm, col_hbm, row_hbm, out_hbm,
                    colv0, colv1, rowv, buf0, buf1, ztile, acc, supc,
                    sem0, sem1):
    c = lax.axis_index("c")
    s = lax.axis_index("s")

    for r in range(8):
        for k in range(HCH // 16):
            ztile[r, pl.ds(k * 16, 16)] = jnp.zeros((16,), jnp.float32)
    base_r = s * ROWS_PER_TILE
    @pl.loop(0, ROWS_PER_TILE // 8)
    def _zero(i):
        pltpu.sync_copy(ztile, acc.at[pl.ds(base_r + i * 8, 8)])

    @pl.loop(0, ROWS_PER_TILE // CHUNK)
    def _stage(i):
        off = base_r + i * CHUNK
        pltpu.sync_copy(sup_hbm.at[c, pl.ds(off, CHUNK)], buf0)
        pltpu.sync_copy(buf0, supc.at[pl.ds(off, CHUNK)])
    plsc.subcore_barrier()

    colvs = (colv0, colv1)
    bufs = (buf0, buf1)
    sems = (sem0, sem1)
    pltpu.sync_copy(col_hbm.at[s, 0], colv0)
    pltpu.async_copy(supc.at[colv0], buf0, sem0)
    @pl.loop(0, N_CHUNKS_T // 2)
    def _pair(i):
        j0 = 2 * i
        for p in range(2):
            j = j0 + p
            jn = lax.rem(j + 1, N_CHUNKS_T)
            pltpu.sync_copy(col_hbm.at[s, jn], colvs[1 - p])
            pltpu.async_copy(supc.at[colvs[1 - p]], bufs[1 - p], sems[1 - p])
            pltpu.sync_copy(row_hbm.at[s, j], rowv.at[p])
            pltpu.make_async_copy(supc.at[colvs[p]], bufs[p], sems[p]).wait()
            pltpu.sync_copy(bufs[p], acc.at[rowv.at[p]], add=True)
    pltpu.make_async_copy(supc.at[colv0], buf0, sem0).wait()
    plsc.subcore_barrier()

    @pl.loop(0, ROWS_PER_TILE // CHUNK)
    def _wb(i):
        off = base_r + i * CHUNK
        pltpu.sync_copy(acc.at[pl.ds(off, CHUNK)], buf0)
        pltpu.sync_copy(buf0, out_hbm.at[c, pl.ds(off, CHUNK)])


def _combine_cs_body(p0_ref, p1_ref, sup_ref, dis_ref, out_ref):
    p = jnp.concatenate([p0_ref[...], p1_ref[...]], axis=1)
    out_ref[...] = dis_ref[...] * (p + sup_ref[...])


def _combine_cs_call(p0, p1, sup, dis):
    return pl.pallas_call(
        _combine_cs_body,
        grid=(GRID,),
        in_specs=[
            pl.BlockSpec((BLK, HCH), lambda i: (i, 0)),
            pl.BlockSpec((BLK, HCH), lambda i: (i, 0)),
            pl.BlockSpec((BLK, OUT_CH), lambda i: (i, 0)),
            pl.BlockSpec((BLK, 1), lambda i: (i, 0)),
        ],
        out_specs=pl.BlockSpec((BLK, OUT_CH), lambda i: (i, 0)),
        out_shape=jax.ShapeDtypeStruct((NPAD, OUT_CH), jnp.float32),
    )(p0, p1, sup, dis)


def _support_body(x_ref, wt_ref, b_ref, h0_ref, h1_ref, sup_ref, dis_ref):
    deg = 1.0 + h0_ref[...] + h1_ref[...]
    dis = lax.rsqrt(deg)
    s = jnp.dot(x_ref[...], wt_ref[...],
                preferred_element_type=jnp.float32) + b_ref[...]
    sup_ref[...] = dis * s
    dis_ref[...] = dis


def _support_call(x_pad, wt, b2, h0, h1):
    return pl.pallas_call(
        _support_body,
        grid=(GRID,),
        in_specs=[
            pl.BlockSpec((BLK, IN_CH), lambda i: (i, 0)),
            pl.BlockSpec((IN_CH, OUT_CH), lambda i: (0, 0)),
            pl.BlockSpec((1, OUT_CH), lambda i: (0, 0)),
            pl.BlockSpec((BLK, 1), lambda i: (i, 0)),
            pl.BlockSpec((BLK, 1), lambda i: (i, 0)),
        ],
        out_specs=[
            pl.BlockSpec((BLK, OUT_CH), lambda i: (i, 0)),
            pl.BlockSpec((BLK, 1), lambda i: (i, 0)),
        ],
        out_shape=[
            jax.ShapeDtypeStruct((NPAD, OUT_CH), jnp.float32),
            jax.ShapeDtypeStruct((NPAD, 1), jnp.float32),
        ],
    )(x_pad, wt, b2, h0, h1)


def _combine_body(p0_ref, p1_ref, sup_ref, dis_ref, out_ref):
    out_ref[...] = dis_ref[...] * (p0_ref[...] + p1_ref[...] + sup_ref[...])


def _combine_call(p0, p1, sup, dis):
    return pl.pallas_call(
        _combine_body,
        grid=(GRID,),
        in_specs=[
            pl.BlockSpec((BLK, OUT_CH), lambda i: (i, 0)),
            pl.BlockSpec((BLK, OUT_CH), lambda i: (i, 0)),
            pl.BlockSpec((BLK, OUT_CH), lambda i: (i, 0)),
            pl.BlockSpec((BLK, 1), lambda i: (i, 0)),
        ],
        out_specs=pl.BlockSpec((BLK, OUT_CH), lambda i: (i, 0)),
        out_shape=jax.ShapeDtypeStruct((NPAD, OUT_CH), jnp.float32),
    )(p0, p1, sup, dis)


def kernel(x, edge_index, W, b):
    ei = edge_index.astype(jnp.int32)
    row = jnp.pad(ei[0], (0, EPAD - N_EDGES), constant_values=N_NODES)
    col = jnp.pad(ei[1], (0, EPAD - N_EDGES), constant_values=0)

    hist = _deg_kernel(row)
    h0 = hist[0].reshape(NPAD, 1)
    h1 = hist[1].reshape(NPAD, 1)

    x_pad = jnp.pad(x, ((0, NPAD - N_NODES), (0, 0)))
    wt = W.T
    b2 = b.reshape(1, OUT_CH)
    sup, dis = _support_call(x_pad, wt, b2, h0, h1)

    supT = sup.reshape(NPAD, NC, HCH).transpose(1, 0, 2)
    col3 = col.reshape(NS, N_CHUNKS_T, CHUNK)
    row3 = row.reshape(NS, N_CHUNKS_T, CHUNK)
    partials = _edge_kernel_cs(supT, col3, row3)
    out = _combine_cs_call(partials[0], partials[1], sup, dis)
    return out[:N_NODES]


# speedup vs baseline: 1.7938x; 1.2477x over previous
import functools

import jax
import jax.numpy as jnp
from jax import lax
from jax.experimental import pallas as pl
from jax.experimental.pallas import tpu as pltpu
from jax.experimental.pallas import tpu_sc as plsc

N_NODES = 10000
IN_CH = 128
OUT_CH = 128

NC = 2
NS = 16
NW = NC * NS
CHUNK = 128
NPAD = 10240
ROWS_PER_TILE = NPAD // NS

N_EDGES = 320000
N_CHUNKS_W = 80
E_PER_W = N_CHUNKS_W * CHUNK
EPAD = E_PER_W * NW
E_PER_C = EPAD // NC

N_CHUNKS_PAIR = 160
NCH0 = 118
NCH1 = N_CHUNKS_PAIR - NCH0

BLK = 1024
GRID = NPAD // BLK

_mesh = lambda: plsc.VectorSubcoreMesh(
    core_axis_name="c", subcore_axis_name="s", num_cores=NC, num_subcores=NS)


@functools.partial(
    pl.kernel,
    out_type=jax.ShapeDtypeStruct((NC, NPAD), jnp.float32),
    mesh=_mesh(),
    scratch_types=[
        pltpu.VMEM((CHUNK,), jnp.int32),
        pltpu.VMEM((CHUNK,), jnp.float32),
        pltpu.VMEM_SHARED((NPAD,), jnp.float32),
    ],
)
def _deg_kernel(row_hbm, hist_hbm, idxv, onesv, acc):
    c = lax.axis_index("c")
    s = lax.axis_index("s")

    for k in range(CHUNK // 16):
        onesv[pl.ds(k * 16, 16)] = jnp.zeros((16,), jnp.float32)
    base_r = s * ROWS_PER_TILE
    @pl.loop(0, ROWS_PER_TILE // CHUNK)
    def _zero(i):
        pltpu.sync_copy(onesv, acc.at[pl.ds(base_r + i * CHUNK, CHUNK)])
    for k in range(CHUNK // 16):
        onesv[pl.ds(k * 16, 16)] = jnp.ones((16,), jnp.float32)
    plsc.subcore_barrier()

    base_e = c * E_PER_C + s * E_PER_W
    @pl.loop(0, N_CHUNKS_W)
    def _hist(j):
        pltpu.sync_copy(row_hbm.at[pl.ds(base_e + j * CHUNK, CHUNK)], idxv)
        pltpu.sync_copy(onesv, acc.at[idxv], add=True)
    plsc.subcore_barrier()

    @pl.loop(0, ROWS_PER_TILE // CHUNK)
    def _wb(i):
        off = base_r + i * CHUNK
        pltpu.sync_copy(acc.at[pl.ds(off, CHUNK)], onesv)
        pltpu.sync_copy(onesv, hist_hbm.at[c, pl.ds(off, CHUNK)])


@functools.partial(
    pl.kernel,
    out_type=jax.ShapeDtypeStruct((NC, NPAD, OUT_CH), jnp.float32),
    mesh=_mesh(),
    scratch_types=[
        pltpu.VMEM((CHUNK,), jnp.int32),
        pltpu.VMEM((CHUNK,), jnp.int32),
        pltpu.VMEM((2, CHUNK), jnp.int32),
        pltpu.VMEM((CHUNK, OUT_CH), jnp.float32),
        pltpu.VMEM((CHUNK, OUT_CH), jnp.float32),
        pltpu.VMEM((8, OUT_CH), jnp.float32),
        pltpu.VMEM_SHARED((NPAD, OUT_CH), jnp.float32),
        pltpu.SemaphoreType.DMA,
        pltpu.SemaphoreType.DMA,
    ],
)
def _edge_kernel(sup_hbm, col_hbm, row_hbm, out_hbm,
                 colv0, colv1, rowv, buf0, buf1, ztile, acc, sem0, sem1):
    c = lax.axis_index("c")
    s = lax.axis_index("s")

    for r in range(8):
        for k in range(OUT_CH // 16):
            ztile[r, pl.ds(k * 16, 16)] = jnp.zeros((16,), jnp.float32)
    base_r = s * ROWS_PER_TILE
    @pl.loop(0, ROWS_PER_TILE // 8)
    def _zero(i):
        pltpu.sync_copy(ztile, acc.at[pl.ds(base_r + i * 8, 8)])
    plsc.subcore_barrier()

    n_my = jnp.where(c == 0, NCH0, NCH1)
    base_e = jnp.where(c == 0, s * NCH0, NS * NCH0 + s * NCH1) * CHUNK

    colvs = (colv0, colv1)
    bufs = (buf0, buf1)
    sems = (sem0, sem1)
    pltpu.sync_copy(col_hbm.at[pl.ds(base_e, CHUNK)], colv0)
    pltpu.async_copy(sup_hbm.at[colv0], buf0, sem0)
    @pl.loop(0, n_my // 2)
    def _pair(i):
        j0 = 2 * i
        for p in range(2):
            j = j0 + p
            jn = lax.rem(j + 1, n_my)
            pltpu.sync_copy(col_hbm.at[pl.ds(base_e + jn * CHUNK, CHUNK)],
                            colvs[1 - p])
            pltpu.async_copy(sup_hbm.at[colvs[1 - p]], bufs[1 - p], sems[1 - p])
            pltpu.sync_copy(row_hbm.at[pl.ds(base_e + j * CHUNK, CHUNK)],
                            rowv.at[p])
            pltpu.make_async_copy(sup_hbm.at[colvs[p]], bufs[p], sems[p]).wait()
            pltpu.sync_copy(bufs[p], acc.at[rowv.at[p]], add=True)
    pltpu.make_async_copy(sup_hbm.at[colv0], buf0, sem0).wait()
    plsc.subcore_barrier()

    @pl.loop(0, ROWS_PER_TILE // CHUNK)
    def _wb(i):
        off = base_r + i * CHUNK
        pltpu.sync_copy(acc.at[pl.ds(off, CHUNK)], buf0)
        pltpu.sync_copy(buf0, out_hbm.at[c, pl.ds(off, CHUNK)])


HCH = OUT_CH // NC
N_CHUNKS_T = EPAD // (NS * CHUNK)
GRP = 8
N_GRP = N_CHUNKS_T // GRP


@functools.partial(
    pl.kernel,
    out_type=jax.ShapeDtypeStruct((NC, NPAD, HCH), jnp.float32),
    mesh=_mesh(),
    compiler_params=pltpu.CompilerParams(use_tc_tiling_on_sc=False),
    scratch_types=[
        pltpu.VMEM((2, GRP, CHUNK), jnp.int32),
        pltpu.VMEM((2, GRP, CHUNK), jnp.int32),
        pltpu.VMEM((CHUNK, HCH), jnp.float32),
        pltpu.VMEM((CHUNK, HCH), jnp.float32),
        pltpu.VMEM((8, HCH), jnp.float32),
        pltpu.VMEM_SHARED((NPAD, HCH), jnp.float32),
        pltpu.VMEM_SHARED((NPAD, HCH), jnp.float32),
        pltpu.SemaphoreType.DMA,
        pltpu.SemaphoreType.DMA,
        pltpu.SemaphoreType.DMA,
    ],
)
def _edge_kernel_cs(sup_hbm, col_hbm, row_hbm, out_hbm,
                    colg, rowg, buf0, buf1, ztile, acc, supc,
                    sem0, sem1, semi):
    c = lax.axis_index("c")
    s = lax.axis_index("s")

    for r in range(8):
        for k in range(HCH // 16):
            ztile[r, pl.ds(k * 16, 16)] = jnp.zeros((16,), jnp.float32)
    base_r = s * ROWS_PER_TILE
    @pl.loop(0, ROWS_PER_TILE // 8)
    def _zero(i):
        pltpu.sync_copy(ztile, acc.at[pl.ds(base_r + i * 8, 8)])

    @pl.loop(0, ROWS_PER_TILE // CHUNK)
    def _stage(i):
        off = base_r + i * CHUNK
        pltpu.sync_copy(sup_hbm.at[c, pl.ds(off, CHUNK)], buf0)
        pltpu.sync_copy(buf0, supc.at[pl.ds(off, CHUNK)])
    plsc.subcore_barrier()

    bufs = (buf0, buf1)
    sems = (sem0, sem1)
    pltpu.sync_copy(col_hbm.at[s, pl.ds(0, GRP)], colg.at[0])
    pltpu.sync_copy(row_hbm.at[s, pl.ds(0, GRP)], rowg.at[0])
    pltpu.async_copy(supc.at[colg.at[0, 0]], buf0, sem0)
    @pl.loop(0, N_GRP)
    def _grp(g):
        q = lax.rem(g, 2)
        qn = 1 - q
        gn = lax.rem(g + 1, N_GRP)
        pltpu.async_copy(col_hbm.at[s, pl.ds(gn * GRP, GRP)], colg.at[qn], semi)
        pltpu.async_copy(row_hbm.at[s, pl.ds(gn * GRP, GRP)], rowg.at[qn], semi)
        for k in range(GRP):
            p = k % 2
            if k < GRP - 1:
                pltpu.async_copy(supc.at[colg.at[q, k + 1]],
                                 bufs[1 - p], sems[1 - p])
            else:
                pltpu.make_async_copy(col_hbm.at[s, pl.ds(0, GRP)],
                                      colg.at[qn], semi).wait()
                pltpu.make_async_copy(row_hbm.at[s, pl.ds(0, GRP)],
                                      rowg.at[qn], semi).wait()
                pltpu.async_copy(supc.at[colg.at[qn, 0]],
                                 bufs[1 - p], sems[1 - p])
            pltpu.make_async_copy(supc.at[colg.at[q, k]],
                                  bufs[p], sems[p]).wait()
            pltpu.sync_copy(bufs[p], acc.at[rowg.at[q, k]], add=True)
    pltpu.make_async_copy(supc.at[colg.at[0, 0]], buf0, sem0).wait()
    plsc.subcore_barrier()

    @pl.loop(0, ROWS_PER_TILE // CHUNK)
    def _wb(i):
        off = base_r + i * CHUNK
        pltpu.sync_copy(acc.at[pl.ds(off, CHUNK)], buf0)
        pltpu.sync_copy(buf0, out_hbm.at[c, pl.ds(off, CHUNK)])


def _combine_cs_body(p0_ref, p1_ref, sup_ref, dis_ref, out_ref):
    p = jnp.concatenate([p0_ref[...], p1_ref[...]], axis=1)
    out_ref[...] = dis_ref[...] * (p + sup_ref[...])


def _combine_cs_call(p0, p1, sup, dis):
    return pl.pallas_call(
        _combine_cs_body,
        grid=(GRID,),
        in_specs=[
            pl.BlockSpec((BLK, HCH), lambda i: (i, 0)),
            pl.BlockSpec((BLK, HCH), lambda i: (i, 0)),
            pl.BlockSpec((BLK, OUT_CH), lambda i: (i, 0)),
            pl.BlockSpec((BLK, 1), lambda i: (i, 0)),
        ],
        out_specs=pl.BlockSpec((BLK, OUT_CH), lambda i: (i, 0)),
        out_shape=jax.ShapeDtypeStruct((NPAD, OUT_CH), jnp.float32),
    )(p0, p1, sup, dis)


def _support_body(x_ref, wt_ref, b_ref, h0_ref, h1_ref, sup_ref, dis_ref):
    deg = 1.0 + h0_ref[...] + h1_ref[...]
    dis = lax.rsqrt(deg)
    s = jnp.dot(x_ref[...], wt_ref[...],
                preferred_element_type=jnp.float32) + b_ref[...]
    sup_ref[...] = dis * s
    dis_ref[...] = dis


def _support_call(x_pad, wt, b2, h0, h1):
    return pl.pallas_call(
        _support_body,
        grid=(GRID,),
        in_specs=[
            pl.BlockSpec((BLK, IN_CH), lambda i: (i, 0)),
            pl.BlockSpec((IN_CH, OUT_CH), lambda i: (0, 0)),
            pl.BlockSpec((1, OUT_CH), lambda i: (0, 0)),
            pl.BlockSpec((BLK, 1), lambda i: (i, 0)),
            pl.BlockSpec((BLK, 1), lambda i: (i, 0)),
        ],
        out_specs=[
            pl.BlockSpec((BLK, OUT_CH), lambda i: (i, 0)),
            pl.BlockSpec((BLK, 1), lambda i: (i, 0)),
        ],
        out_shape=[
            jax.ShapeDtypeStruct((NPAD, OUT_CH), jnp.float32),
            jax.ShapeDtypeStruct((NPAD, 1), jnp.float32),
        ],
    )(x_pad, wt, b2, h0, h1)


def _combine_body(p0_ref, p1_ref, sup_ref, dis_ref, out_ref):
    out_ref[...] = dis_ref[...] * (p0_ref[...] + p1_ref[...] + sup_ref[...])


def _combine_call(p0, p1, sup, dis):
    return pl.pallas_call(
        _combine_body,
        grid=(GRID,),
        in_specs=[
            pl.BlockSpec((BLK, OUT_CH), lambda i: (i, 0)),
            pl.BlockSpec((BLK, OUT_CH), lambda i: (i, 0)),
            pl.BlockSpec((BLK, OUT_CH), lambda i: (i, 0)),
            pl.BlockSpec((BLK, 1), lambda i: (i, 0)),
        ],
        out_specs=pl.BlockSpec((BLK, OUT_CH), lambda i: (i, 0)),
        out_shape=jax.ShapeDtypeStruct((NPAD, OUT_CH), jnp.float32),
    )(p0, p1, sup, dis)


def kernel(x, edge_index, W, b):
    ei = edge_index.astype(jnp.int32)
    row = jnp.pad(ei[0], (0, EPAD - N_EDGES), constant_values=N_NODES)
    col = jnp.pad(ei[1], (0, EPAD - N_EDGES), constant_values=0)

    hist = _deg_kernel(row)
    h0 = hist[0].reshape(NPAD, 1)
    h1 = hist[1].reshape(NPAD, 1)

    x_pad = jnp.pad(x, ((0, NPAD - N_NODES), (0, 0)))
    wt = W.T
    b2 = b.reshape(1, OUT_CH)
    sup, dis = _support_call(x_pad, wt, b2, h0, h1)

    supT = sup.reshape(NPAD, NC, HCH).transpose(1, 0, 2)
    col3 = col.reshape(NS, N_CHUNKS_T, CHUNK)
    row3 = row.reshape(NS, N_CHUNKS_T, CHUNK)
    partials = _edge_kernel_cs(supT, col3, row3)
    out = _combine_cs_call(partials[0], partials[1], sup, dis)
    return out[:N_NODES]


# speedup vs baseline: 2.0281x; 1.1306x over previous
import functools

import jax
import jax.numpy as jnp
from jax import lax
from jax.experimental import pallas as pl
from jax.experimental.pallas import tpu as pltpu
from jax.experimental.pallas import tpu_sc as plsc

N_NODES = 10000
IN_CH = 128
OUT_CH = 128

NC = 2
NS = 16
NW = NC * NS
CHUNK = 128
NPAD = 10240
ROWS_PER_TILE = NPAD // NS

N_EDGES = 320000
N_CHUNKS_W = 80
E_PER_W = N_CHUNKS_W * CHUNK
EPAD = E_PER_W * NW
E_PER_C = EPAD // NC

N_CHUNKS_PAIR = 160
NCH0 = 118
NCH1 = N_CHUNKS_PAIR - NCH0

BLK = 1024
GRID = NPAD // BLK

_mesh = lambda: plsc.VectorSubcoreMesh(
    core_axis_name="c", subcore_axis_name="s", num_cores=NC, num_subcores=NS)


@functools.partial(
    pl.kernel,
    out_type=jax.ShapeDtypeStruct((NC, NPAD), jnp.float32),
    mesh=_mesh(),
    scratch_types=[
        pltpu.VMEM((N_CHUNKS_W, CHUNK), jnp.int32),
        pltpu.VMEM((CHUNK,), jnp.float32),
        pltpu.VMEM_SHARED((NPAD,), jnp.float32),
        pltpu.SemaphoreType.DMA,
    ],
)
def _deg_kernel(row_hbm, hist_hbm, rowst, onesv, acc, sem):
    c = lax.axis_index("c")
    s = lax.axis_index("s")
    wid = c * NS + s

    for k in range(CHUNK // 16):
        onesv[pl.ds(k * 16, 16)] = jnp.zeros((16,), jnp.float32)
    base_r = s * ROWS_PER_TILE
    @pl.loop(0, ROWS_PER_TILE // CHUNK)
    def _zero(i):
        pltpu.sync_copy(onesv, acc.at[pl.ds(base_r + i * CHUNK, CHUNK)])
    pltpu.sync_copy(row_hbm.at[wid], rowst)
    for k in range(CHUNK // 16):
        onesv[pl.ds(k * 16, 16)] = jnp.ones((16,), jnp.float32)
    plsc.subcore_barrier()

    @pl.loop(0, N_CHUNKS_W // 8)
    def _hist(g):
        for k in range(8):
            pltpu.async_copy(onesv, acc.at[rowst.at[g * 8 + k]], sem, add=True)
        for k in range(8):
            pltpu.make_async_copy(onesv, acc.at[rowst.at[0]], sem).wait()
    plsc.subcore_barrier()

    @pl.loop(0, ROWS_PER_TILE // CHUNK)
    def _wb(i):
        off = base_r + i * CHUNK
        pltpu.sync_copy(acc.at[pl.ds(off, CHUNK)], onesv)
        pltpu.sync_copy(onesv, hist_hbm.at[c, pl.ds(off, CHUNK)])


@functools.partial(
    pl.kernel,
    out_type=jax.ShapeDtypeStruct((NC, NPAD, OUT_CH), jnp.float32),
    mesh=_mesh(),
    scratch_types=[
        pltpu.VMEM((CHUNK,), jnp.int32),
        pltpu.VMEM((CHUNK,), jnp.int32),
        pltpu.VMEM((2, CHUNK), jnp.int32),
        pltpu.VMEM((CHUNK, OUT_CH), jnp.float32),
        pltpu.VMEM((CHUNK, OUT_CH), jnp.float32),
        pltpu.VMEM((8, OUT_CH), jnp.float32),
        pltpu.VMEM_SHARED((NPAD, OUT_CH), jnp.float32),
        pltpu.SemaphoreType.DMA,
        pltpu.SemaphoreType.DMA,
    ],
)
def _edge_kernel(sup_hbm, col_hbm, row_hbm, out_hbm,
                 colv0, colv1, rowv, buf0, buf1, ztile, acc, sem0, sem1):
    c = lax.axis_index("c")
    s = lax.axis_index("s")

    for r in range(8):
        for k in range(OUT_CH // 16):
            ztile[r, pl.ds(k * 16, 16)] = jnp.zeros((16,), jnp.float32)
    base_r = s * ROWS_PER_TILE
    @pl.loop(0, ROWS_PER_TILE // 8)
    def _zero(i):
        pltpu.sync_copy(ztile, acc.at[pl.ds(base_r + i * 8, 8)])
    plsc.subcore_barrier()

    n_my = jnp.where(c == 0, NCH0, NCH1)
    base_e = jnp.where(c == 0, s * NCH0, NS * NCH0 + s * NCH1) * CHUNK

    colvs = (colv0, colv1)
    bufs = (buf0, buf1)
    sems = (sem0, sem1)
    pltpu.sync_copy(col_hbm.at[pl.ds(base_e, CHUNK)], colv0)
    pltpu.async_copy(sup_hbm.at[colv0], buf0, sem0)
    @pl.loop(0, n_my // 2)
    def _pair(i):
        j0 = 2 * i
        for p in range(2):
            j = j0 + p
            jn = lax.rem(j + 1, n_my)
            pltpu.sync_copy(col_hbm.at[pl.ds(base_e + jn * CHUNK, CHUNK)],
                            colvs[1 - p])
            pltpu.async_copy(sup_hbm.at[colvs[1 - p]], bufs[1 - p], sems[1 - p])
            pltpu.sync_copy(row_hbm.at[pl.ds(base_e + j * CHUNK, CHUNK)],
                            rowv.at[p])
            pltpu.make_async_copy(sup_hbm.at[colvs[p]], bufs[p], sems[p]).wait()
            pltpu.sync_copy(bufs[p], acc.at[rowv.at[p]], add=True)
    pltpu.make_async_copy(sup_hbm.at[colv0], buf0, sem0).wait()
    plsc.subcore_barrier()

    @pl.loop(0, ROWS_PER_TILE // CHUNK)
    def _wb(i):
        off = base_r + i * CHUNK
        pltpu.sync_copy(acc.at[pl.ds(off, CHUNK)], buf0)
        pltpu.sync_copy(buf0, out_hbm.at[c, pl.ds(off, CHUNK)])


HCH = OUT_CH // NC
N_CHUNKS_T = EPAD // (NS * CHUNK)
GRP = 8
N_GRP = N_CHUNKS_T // GRP


@functools.partial(
    pl.kernel,
    out_type=jax.ShapeDtypeStruct((NC, NPAD, HCH), jnp.float32),
    mesh=_mesh(),
    compiler_params=pltpu.CompilerParams(use_tc_tiling_on_sc=False),
    scratch_types=[
        pltpu.VMEM((2, GRP, CHUNK), jnp.int32),
        pltpu.VMEM((2, GRP, CHUNK), jnp.int32),
        pltpu.VMEM((CHUNK, HCH), jnp.float32),
        pltpu.VMEM((CHUNK, HCH), jnp.float32),
        pltpu.VMEM((8, HCH), jnp.float32),
        pltpu.VMEM_SHARED((NPAD, HCH), jnp.float32),
        pltpu.VMEM_SHARED((NPAD, HCH), jnp.float32),
        pltpu.SemaphoreType.DMA,
        pltpu.SemaphoreType.DMA,
        pltpu.SemaphoreType.DMA,
    ],
)
def _edge_kernel_cs(sup_hbm, col_hbm, row_hbm, out_hbm,
                    colg, rowg, buf0, buf1, ztile, acc, supc,
                    sem0, sem1, semi):
    c = lax.axis_index("c")
    s = lax.axis_index("s")

    for r in range(8):
        for k in range(HCH // 16):
            ztile[r, pl.ds(k * 16, 16)] = jnp.zeros((16,), jnp.float32)
    base_r = s * ROWS_PER_TILE
    @pl.loop(0, ROWS_PER_TILE // 8)
    def _zero(i):
        pltpu.sync_copy(ztile, acc.at[pl.ds(base_r + i * 8, 8)])

    @pl.loop(0, ROWS_PER_TILE // CHUNK)
    def _stage(i):
        off = base_r + i * CHUNK
        pltpu.sync_copy(sup_hbm.at[c, pl.ds(off, CHUNK)], buf0)
        pltpu.sync_copy(buf0, supc.at[pl.ds(off, CHUNK)])
    plsc.subcore_barrier()

    bufs = (buf0, buf1)
    sems = (sem0, sem1)
    pltpu.sync_copy(col_hbm.at[s, pl.ds(0, GRP)], colg.at[0])
    pltpu.sync_copy(row_hbm.at[s, pl.ds(0, GRP)], rowg.at[0])
    pltpu.async_copy(supc.at[colg.at[0, 0]], buf0, sem0)
    @pl.loop(0, N_GRP)
    def _grp(g):
        q = lax.rem(g, 2)
        qn = 1 - q
        gn = lax.rem(g + 1, N_GRP)
        pltpu.async_copy(col_hbm.at[s, pl.ds(gn * GRP, GRP)], colg.at[qn], semi)
        pltpu.async_copy(row_hbm.at[s, pl.ds(gn * GRP, GRP)], rowg.at[qn], semi)
        for k in range(GRP):
            p = k % 2
            if k < GRP - 1:
                pltpu.async_copy(supc.at[colg.at[q, k + 1]],
                                 bufs[1 - p], sems[1 - p])
            else:
                pltpu.make_async_copy(col_hbm.at[s, pl.ds(0, GRP)],
                                      colg.at[qn], semi).wait()
                pltpu.make_async_copy(row_hbm.at[s, pl.ds(0, GRP)],
                                      rowg.at[qn], semi).wait()
                pltpu.async_copy(supc.at[colg.at[qn, 0]],
                                 bufs[1 - p], sems[1 - p])
            pltpu.make_async_copy(supc.at[colg.at[q, k]],
                                  bufs[p], sems[p]).wait()
            pltpu.sync_copy(bufs[p], acc.at[rowg.at[q, k]], add=True)
    pltpu.make_async_copy(supc.at[colg.at[0, 0]], buf0, sem0).wait()
    plsc.subcore_barrier()

    @pl.loop(0, ROWS_PER_TILE // CHUNK)
    def _wb(i):
        off = base_r + i * CHUNK
        pltpu.sync_copy(acc.at[pl.ds(off, CHUNK)], buf0)
        pltpu.sync_copy(buf0, out_hbm.at[c, pl.ds(off, CHUNK)])


def _combine_cs_body(p0_ref, p1_ref, sup_ref, dis_ref, out_ref):
    p = jnp.concatenate([p0_ref[...], p1_ref[...]], axis=1)
    out_ref[...] = dis_ref[...] * (p + sup_ref[...])


def _combine_cs_call(p0, p1, sup, dis):
    return pl.pallas_call(
        _combine_cs_body,
        grid=(GRID,),
        in_specs=[
            pl.BlockSpec((BLK, HCH), lambda i: (i, 0)),
            pl.BlockSpec((BLK, HCH), lambda i: (i, 0)),
            pl.BlockSpec((BLK, OUT_CH), lambda i: (i, 0)),
            pl.BlockSpec((BLK, 1), lambda i: (i, 0)),
        ],
        out_specs=pl.BlockSpec((BLK, OUT_CH), lambda i: (i, 0)),
        out_shape=jax.ShapeDtypeStruct((NPAD, OUT_CH), jnp.float32),
    )(p0, p1, sup, dis)


def _support_body(x_ref, wt_ref, b_ref, h0_ref, h1_ref, sup_ref, dis_ref):
    deg = 1.0 + h0_ref[...] + h1_ref[...]
    dis = lax.rsqrt(deg)
    s = jnp.dot(x_ref[...], wt_ref[...],
                preferred_element_type=jnp.float32) + b_ref[...]
    sup_ref[...] = dis * s
    dis_ref[...] = dis


def _support_call(x_pad, wt, b2, h0, h1):
    return pl.pallas_call(
        _support_body,
        grid=(GRID,),
        in_specs=[
            pl.BlockSpec((BLK, IN_CH), lambda i: (i, 0)),
            pl.BlockSpec((IN_CH, OUT_CH), lambda i: (0, 0)),
            pl.BlockSpec((1, OUT_CH), lambda i: (0, 0)),
            pl.BlockSpec((BLK, 1), lambda i: (i, 0)),
            pl.BlockSpec((BLK, 1), lambda i: (i, 0)),
        ],
        out_specs=[
            pl.BlockSpec((BLK, OUT_CH), lambda i: (i, 0)),
            pl.BlockSpec((BLK, 1), lambda i: (i, 0)),
        ],
        out_shape=[
            jax.ShapeDtypeStruct((NPAD, OUT_CH), jnp.float32),
            jax.ShapeDtypeStruct((NPAD, 1), jnp.float32),
        ],
    )(x_pad, wt, b2, h0, h1)


def _combine_body(p0_ref, p1_ref, sup_ref, dis_ref, out_ref):
    out_ref[...] = dis_ref[...] * (p0_ref[...] + p1_ref[...] + sup_ref[...])


def _combine_call(p0, p1, sup, dis):
    return pl.pallas_call(
        _combine_body,
        grid=(GRID,),
        in_specs=[
            pl.BlockSpec((BLK, OUT_CH), lambda i: (i, 0)),
            pl.BlockSpec((BLK, OUT_CH), lambda i: (i, 0)),
            pl.BlockSpec((BLK, OUT_CH), lambda i: (i, 0)),
            pl.BlockSpec((BLK, 1), lambda i: (i, 0)),
        ],
        out_specs=pl.BlockSpec((BLK, OUT_CH), lambda i: (i, 0)),
        out_shape=jax.ShapeDtypeStruct((NPAD, OUT_CH), jnp.float32),
    )(p0, p1, sup, dis)


def kernel(x, edge_index, W, b):
    ei = edge_index.astype(jnp.int32)
    row = jnp.pad(ei[0], (0, EPAD - N_EDGES), constant_values=N_NODES)
    col = jnp.pad(ei[1], (0, EPAD - N_EDGES), constant_values=0)

    hist = _deg_kernel(row.reshape(NW, N_CHUNKS_W, CHUNK))
    h0 = hist[0].reshape(NPAD, 1)
    h1 = hist[1].reshape(NPAD, 1)

    x_pad = jnp.pad(x, ((0, NPAD - N_NODES), (0, 0)))
    wt = W.T
    b2 = b.reshape(1, OUT_CH)
    sup, dis = _support_call(x_pad, wt, b2, h0, h1)

    supT = sup.reshape(NPAD, NC, HCH).transpose(1, 0, 2)
    col3 = col.reshape(NS, N_CHUNKS_T, CHUNK)
    row3 = row.reshape(NS, N_CHUNKS_T, CHUNK)
    partials = _edge_kernel_cs(supT, col3, row3)
    out = _combine_cs_call(partials[0], partials[1], sup, dis)
    return out[:N_NODES]


# speedup vs baseline: 2.1321x; 1.0513x over previous
import functools

import jax
import jax.numpy as jnp
from jax import lax
from jax.experimental import pallas as pl
from jax.experimental.pallas import tpu as pltpu
from jax.experimental.pallas import tpu_sc as plsc

N_NODES = 10000
IN_CH = 128
OUT_CH = 128

NC = 2
NS = 16
NW = NC * NS
CHUNK = 128
NPAD = 10240
ROWS_PER_TILE = NPAD // NS

N_EDGES = 320000
N_CHUNKS_W = 80
E_PER_W = N_CHUNKS_W * CHUNK
EPAD = E_PER_W * NW
E_PER_C = EPAD // NC

N_CHUNKS_PAIR = 160
NCH0 = 118
NCH1 = N_CHUNKS_PAIR - NCH0

BLK = 1024
GRID = NPAD // BLK

_mesh = lambda: plsc.VectorSubcoreMesh(
    core_axis_name="c", subcore_axis_name="s", num_cores=NC, num_subcores=NS)


@functools.partial(
    pl.kernel,
    out_type=jax.ShapeDtypeStruct((NC, NPAD), jnp.float32),
    mesh=_mesh(),
    scratch_types=[
        pltpu.VMEM((N_CHUNKS_W, CHUNK), jnp.int32),
        pltpu.VMEM((CHUNK,), jnp.float32),
        pltpu.VMEM_SHARED((NPAD,), jnp.float32),
        pltpu.SemaphoreType.DMA,
    ],
)
def _deg_kernel(row_hbm, hist_hbm, rowst, onesv, acc, sem):
    c = lax.axis_index("c")
    s = lax.axis_index("s")
    wid = c * NS + s

    for k in range(CHUNK // 16):
        onesv[pl.ds(k * 16, 16)] = jnp.zeros((16,), jnp.float32)
    base_r = s * ROWS_PER_TILE
    @pl.loop(0, ROWS_PER_TILE // CHUNK)
    def _zero(i):
        pltpu.sync_copy(onesv, acc.at[pl.ds(base_r + i * CHUNK, CHUNK)])
    pltpu.sync_copy(row_hbm.at[wid], rowst)
    for k in range(CHUNK // 16):
        onesv[pl.ds(k * 16, 16)] = jnp.ones((16,), jnp.float32)
    plsc.subcore_barrier()

    @pl.loop(0, N_CHUNKS_W // 8)
    def _hist(g):
        for k in range(8):
            pltpu.async_copy(onesv, acc.at[rowst.at[g * 8 + k]], sem, add=True)
        for k in range(8):
            pltpu.make_async_copy(onesv, acc.at[rowst.at[0]], sem).wait()
    plsc.subcore_barrier()

    @pl.loop(0, ROWS_PER_TILE // CHUNK)
    def _wb(i):
        off = base_r + i * CHUNK
        pltpu.sync_copy(acc.at[pl.ds(off, CHUNK)], onesv)
        pltpu.sync_copy(onesv, hist_hbm.at[c, pl.ds(off, CHUNK)])


@functools.partial(
    pl.kernel,
    out_type=jax.ShapeDtypeStruct((NC, NPAD, OUT_CH), jnp.float32),
    mesh=_mesh(),
    scratch_types=[
        pltpu.VMEM((CHUNK,), jnp.int32),
        pltpu.VMEM((CHUNK,), jnp.int32),
        pltpu.VMEM((2, CHUNK), jnp.int32),
        pltpu.VMEM((CHUNK, OUT_CH), jnp.float32),
        pltpu.VMEM((CHUNK, OUT_CH), jnp.float32),
        pltpu.VMEM((8, OUT_CH), jnp.float32),
        pltpu.VMEM_SHARED((NPAD, OUT_CH), jnp.float32),
        pltpu.SemaphoreType.DMA,
        pltpu.SemaphoreType.DMA,
    ],
)
def _edge_kernel(sup_hbm, col_hbm, row_hbm, out_hbm,
                 colv0, colv1, rowv, buf0, buf1, ztile, acc, sem0, sem1):
    c = lax.axis_index("c")
    s = lax.axis_index("s")

    for r in range(8):
        for k in range(OUT_CH // 16):
            ztile[r, pl.ds(k * 16, 16)] = jnp.zeros((16,), jnp.float32)
    base_r = s * ROWS_PER_TILE
    @pl.loop(0, ROWS_PER_TILE // 8)
    def _zero(i):
        pltpu.sync_copy(ztile, acc.at[pl.ds(base_r + i * 8, 8)])
    plsc.subcore_barrier()

    n_my = jnp.where(c == 0, NCH0, NCH1)
    base_e = jnp.where(c == 0, s * NCH0, NS * NCH0 + s * NCH1) * CHUNK

    colvs = (colv0, colv1)
    bufs = (buf0, buf1)
    sems = (sem0, sem1)
    pltpu.sync_copy(col_hbm.at[pl.ds(base_e, CHUNK)], colv0)
    pltpu.async_copy(sup_hbm.at[colv0], buf0, sem0)
    @pl.loop(0, n_my // 2)
    def _pair(i):
        j0 = 2 * i
        for p in range(2):
            j = j0 + p
            jn = lax.rem(j + 1, n_my)
            pltpu.sync_copy(col_hbm.at[pl.ds(base_e + jn * CHUNK, CHUNK)],
                            colvs[1 - p])
            pltpu.async_copy(sup_hbm.at[colvs[1 - p]], bufs[1 - p], sems[1 - p])
            pltpu.sync_copy(row_hbm.at[pl.ds(base_e + j * CHUNK, CHUNK)],
                            rowv.at[p])
            pltpu.make_async_copy(sup_hbm.at[colvs[p]], bufs[p], sems[p]).wait()
            pltpu.sync_copy(bufs[p], acc.at[rowv.at[p]], add=True)
    pltpu.make_async_copy(sup_hbm.at[colv0], buf0, sem0).wait()
    plsc.subcore_barrier()

    @pl.loop(0, ROWS_PER_TILE // CHUNK)
    def _wb(i):
        off = base_r + i * CHUNK
        pltpu.sync_copy(acc.at[pl.ds(off, CHUNK)], buf0)
        pltpu.sync_copy(buf0, out_hbm.at[c, pl.ds(off, CHUNK)])


HCH = OUT_CH // NC
N_CHUNKS_T = EPAD // (NS * CHUNK)
GRP = 8
N_GRP = N_CHUNKS_T // GRP


@functools.partial(
    pl.kernel,
    out_type=jax.ShapeDtypeStruct((NC, NPAD, HCH), jnp.float32),
    mesh=_mesh(),
    compiler_params=pltpu.CompilerParams(use_tc_tiling_on_sc=False),
    scratch_types=[
        pltpu.VMEM((2, GRP, CHUNK), jnp.int32),
        pltpu.VMEM((2, GRP, CHUNK), jnp.int32),
        pltpu.VMEM((CHUNK, HCH), jnp.float32),
        pltpu.VMEM((CHUNK, HCH), jnp.float32),
        pltpu.VMEM((8, HCH), jnp.float32),
        pltpu.VMEM_SHARED((NPAD, HCH), jnp.float32),
        pltpu.VMEM_SHARED((NPAD, HCH), jnp.float32),
        pltpu.SemaphoreType.DMA,
        pltpu.SemaphoreType.DMA,
        pltpu.SemaphoreType.DMA,
    ],
)
def _edge_kernel_cs(sup_hbm, col_hbm, row_hbm, out_hbm,
                    colg, rowg, buf0, buf1, ztile, acc, supc,
                    sem0, sem1, semi):
    c = lax.axis_index("c")
    s = lax.axis_index("s")

    for r in range(8):
        for k in range(HCH // 16):
            ztile[r, pl.ds(k * 16, 16)] = jnp.zeros((16,), jnp.float32)
    base_r = s * ROWS_PER_TILE
    @pl.loop(0, ROWS_PER_TILE // 8)
    def _zero(i):
        pltpu.sync_copy(ztile, acc.at[pl.ds(base_r + i * 8, 8)])

    @pl.loop(0, ROWS_PER_TILE // CHUNK)
    def _stage(i):
        off = base_r + i * CHUNK
        pltpu.sync_copy(sup_hbm.at[c, pl.ds(off, CHUNK)], buf0)
        pltpu.sync_copy(buf0, supc.at[pl.ds(off, CHUNK)])
    plsc.subcore_barrier()

    bufs = (buf0, buf1)
    sems = (sem0, sem1)
    pltpu.sync_copy(col_hbm.at[s, pl.ds(0, GRP)], colg.at[0])
    pltpu.sync_copy(row_hbm.at[s, pl.ds(0, GRP)], rowg.at[0])
    pltpu.async_copy(supc.at[colg.at[0, 0]], buf0, sem0)
    @pl.loop(0, N_GRP)
    def _grp(g):
        q = lax.rem(g, 2)
        qn = 1 - q
        gn = lax.rem(g + 1, N_GRP)
        pltpu.async_copy(col_hbm.at[s, pl.ds(gn * GRP, GRP)], colg.at[qn], semi)
        pltpu.async_copy(row_hbm.at[s, pl.ds(gn * GRP, GRP)], rowg.at[qn], semi)
        for k in range(GRP):
            p = k % 2
            if k < GRP - 1:
                pltpu.async_copy(supc.at[colg.at[q, k + 1]],
                                 bufs[1 - p], sems[1 - p])
            else:
                pltpu.make_async_copy(col_hbm.at[s, pl.ds(0, GRP)],
                                      colg.at[qn], semi).wait()
                pltpu.make_async_copy(row_hbm.at[s, pl.ds(0, GRP)],
                                      rowg.at[qn], semi).wait()
                pltpu.async_copy(supc.at[colg.at[qn, 0]],
                                 bufs[1 - p], sems[1 - p])
            pltpu.make_async_copy(supc.at[colg.at[q, k]],
                                  bufs[p], sems[p]).wait()
            pltpu.sync_copy(bufs[p], acc.at[rowg.at[q, k]], add=True)
    pltpu.make_async_copy(supc.at[colg.at[0, 0]], buf0, sem0).wait()
    plsc.subcore_barrier()

    @pl.loop(0, ROWS_PER_TILE // CHUNK)
    def _wb(i):
        off = base_r + i * CHUNK
        pltpu.sync_copy(acc.at[pl.ds(off, CHUNK)], buf0)
        pltpu.sync_copy(buf0, out_hbm.at[c, pl.ds(off, CHUNK)])


def _combine_cs_body(p_ref, sup_ref, dis_ref, out_ref):
    p = jnp.concatenate([p_ref[0] + sup_ref[0], p_ref[1] + sup_ref[1]], axis=1)
    out_ref[...] = dis_ref[...] * p


def _combine_cs_call(p, supT, dis):
    return pl.pallas_call(
        _combine_cs_body,
        grid=(GRID,),
        in_specs=[
            pl.BlockSpec((NC, BLK, HCH), lambda i: (0, i, 0)),
            pl.BlockSpec((NC, BLK, HCH), lambda i: (0, i, 0)),
            pl.BlockSpec((BLK, 1), lambda i: (i, 0)),
        ],
        out_specs=pl.BlockSpec((BLK, OUT_CH), lambda i: (i, 0)),
        out_shape=jax.ShapeDtypeStruct((NPAD, OUT_CH), jnp.float32),
    )(p, supT, dis)


def _support_body(x_ref, wt_ref, b_ref, h0_ref, h1_ref, sup_ref, dis_ref):
    deg = 1.0 + h0_ref[...] + h1_ref[...]
    dis = lax.rsqrt(deg)
    s = jnp.dot(x_ref[...], wt_ref[...],
                preferred_element_type=jnp.float32) + b_ref[...]
    s2 = dis * s
    sup_ref[0] = s2[:, :HCH]
    sup_ref[1] = s2[:, HCH:]
    dis_ref[...] = dis


def _support_call(x_pad, wt, b2, h0, h1):
    return pl.pallas_call(
        _support_body,
        grid=(GRID,),
        in_specs=[
            pl.BlockSpec((BLK, IN_CH), lambda i: (i, 0)),
            pl.BlockSpec((IN_CH, OUT_CH), lambda i: (0, 0)),
            pl.BlockSpec((1, OUT_CH), lambda i: (0, 0)),
            pl.BlockSpec((BLK, 1), lambda i: (i, 0)),
            pl.BlockSpec((BLK, 1), lambda i: (i, 0)),
        ],
        out_specs=[
            pl.BlockSpec((NC, BLK, HCH), lambda i: (0, i, 0)),
            pl.BlockSpec((BLK, 1), lambda i: (i, 0)),
        ],
        out_shape=[
            jax.ShapeDtypeStruct((NC, NPAD, HCH), jnp.float32),
            jax.ShapeDtypeStruct((NPAD, 1), jnp.float32),
        ],
    )(x_pad, wt, b2, h0, h1)


def _combine_body(p0_ref, p1_ref, sup_ref, dis_ref, out_ref):
    out_ref[...] = dis_ref[...] * (p0_ref[...] + p1_ref[...] + sup_ref[...])


def _combine_call(p0, p1, sup, dis):
    return pl.pallas_call(
        _combine_body,
        grid=(GRID,),
        in_specs=[
            pl.BlockSpec((BLK, OUT_CH), lambda i: (i, 0)),
            pl.BlockSpec((BLK, OUT_CH), lambda i: (i, 0)),
            pl.BlockSpec((BLK, OUT_CH), lambda i: (i, 0)),
            pl.BlockSpec((BLK, 1), lambda i: (i, 0)),
        ],
        out_specs=pl.BlockSpec((BLK, OUT_CH), lambda i: (i, 0)),
        out_shape=jax.ShapeDtypeStruct((NPAD, OUT_CH), jnp.float32),
    )(p0, p1, sup, dis)


def kernel(x, edge_index, W, b):
    ei = edge_index.astype(jnp.int32)
    row = jnp.pad(ei[0], (0, EPAD - N_EDGES), constant_values=N_NODES)
    col = jnp.pad(ei[1], (0, EPAD - N_EDGES), constant_values=0)

    hist = _deg_kernel(row.reshape(NW, N_CHUNKS_W, CHUNK))
    h0 = hist[0].reshape(NPAD, 1)
    h1 = hist[1].reshape(NPAD, 1)

    x_pad = jnp.pad(x, ((0, NPAD - N_NODES), (0, 0)))
    wt = W.T
    b2 = b.reshape(1, OUT_CH)
    supT, dis = _support_call(x_pad, wt, b2, h0, h1)

    col3 = col.reshape(NS, N_CHUNKS_T, CHUNK)
    row3 = row.reshape(NS, N_CHUNKS_T, CHUNK)
    partials = _edge_kernel_cs(supT, col3, row3)
    out = _combine_cs_call(partials, supT, dis)
    return out[:N_NODES]
